# baseline (speedup 1.0000x reference)
import jax
import jax.numpy as jnp
from jax.experimental import pallas as pl
from jax.experimental.pallas import tpu as pltpu

_EPS = 1e-12
_BS = 512


def kernel(concat_embeddings, concat_type, pos_table, gamma, beta):
    del concat_type
    B, S, H = concat_embeddings.shape
    bs = _BS

    def outer(x_hbm, pos_hbm, gamma_ref, beta_ref, o_hbm):
        def inner(x_ref, pos_ref, o_ref):
            x = x_ref[...] + pos_ref[...][None, :, :]
            mean = jnp.mean(x, axis=-1, keepdims=True)
            cent = x - mean
            var = jnp.mean(cent * cent, axis=-1, keepdims=True)
            xhat = cent * jax.lax.rsqrt(var + _EPS)
            o_ref[...] = xhat * gamma_ref[...] + beta_ref[...]

        buf = pl.Buffered(buffer_count=4)
        pipe = pltpu.emit_pipeline(
            inner,
            grid=(S // bs,),
            in_specs=[
                pl.BlockSpec((B, bs, H), lambda s: (0, s, 0), pipeline_mode=buf),
                pl.BlockSpec((bs, H), lambda s: (s, 0), pipeline_mode=buf),
            ],
            out_specs=[
                pl.BlockSpec((B, bs, H), lambda s: (0, s, 0),
                             pipeline_mode=pl.Buffered(buffer_count=2)),
            ],
        )
        pipe(x_hbm, pos_hbm, o_hbm)

    return pl.pallas_call(
        outer,
        in_specs=[
            pl.BlockSpec(memory_space=pl.ANY),
            pl.BlockSpec(memory_space=pl.ANY),
            pl.BlockSpec(memory_space=pltpu.VMEM),
            pl.BlockSpec(memory_space=pltpu.VMEM),
        ],
        out_specs=pl.BlockSpec(memory_space=pl.ANY),
        out_shape=jax.ShapeDtypeStruct((B, S, H), concat_embeddings.dtype),
        compiler_params=pltpu.CompilerParams(
            vmem_limit_bytes=128 * 1024 * 1024),
    )(concat_embeddings, pos_table, gamma, beta)


# speedup vs baseline: 1.0071x; 1.0071x over previous
import jax
import jax.numpy as jnp
from jax.experimental import pallas as pl
from jax.experimental.pallas import tpu as pltpu

_EPS = 1e-12
_BS = 128


def kernel(concat_embeddings, concat_type, pos_table, gamma, beta):
    del concat_type
    B, S, H = concat_embeddings.shape
    bs = _BS

    def outer(x_hbm, pos_hbm, gamma_ref, beta_ref, o_hbm):
        def inner(x_ref, pos_ref, o_ref):
            x = x_ref[...] + pos_ref[...][None, :, :]
            mean = jnp.mean(x, axis=-1, keepdims=True)
            cent = x - mean
            var = jnp.mean(cent * cent, axis=-1, keepdims=True)
            xhat = cent * jax.lax.rsqrt(var + _EPS)
            o_ref[...] = xhat * gamma_ref[...] + beta_ref[...]

        buf = pl.Buffered(buffer_count=6)
        pipe = pltpu.emit_pipeline(
            inner,
            grid=(S // bs,),
            in_specs=[
                pl.BlockSpec((B, bs, H), lambda s: (0, s, 0), pipeline_mode=buf),
                pl.BlockSpec((bs, H), lambda s: (s, 0), pipeline_mode=buf),
            ],
            out_specs=[
                pl.BlockSpec((B, bs, H), lambda s: (0, s, 0),
                             pipeline_mode=pl.Buffered(buffer_count=2)),
            ],
        )
        pipe(x_hbm, pos_hbm, o_hbm)

    return pl.pallas_call(
        outer,
        in_specs=[
            pl.BlockSpec(memory_space=pl.ANY),
            pl.BlockSpec(memory_space=pl.ANY),
            pl.BlockSpec(memory_space=pltpu.VMEM),
            pl.BlockSpec(memory_space=pltpu.VMEM),
        ],
        out_specs=pl.BlockSpec(memory_space=pl.ANY),
        out_shape=jax.ShapeDtypeStruct((B, S, H), concat_embeddings.dtype),
        compiler_params=pltpu.CompilerParams(
            vmem_limit_bytes=128 * 1024 * 1024),
    )(concat_embeddings, pos_table, gamma, beta)


# speedup vs baseline: 1.0105x; 1.0034x over previous
import jax
import jax.numpy as jnp
from jax.experimental import pallas as pl
from jax.experimental.pallas import tpu as pltpu

_EPS = 1e-12
_BS = 256


def kernel(concat_embeddings, concat_type, pos_table, gamma, beta):
    del concat_type
    B, S, H = concat_embeddings.shape
    bs = _BS

    def outer(x_hbm, pos_hbm, gamma_ref, beta_ref, o_hbm):
        def inner(x_ref, pos_ref, o_ref):
            x = x_ref[...] + pos_ref[...][None, :, :]
            mean = jnp.mean(x, axis=-1, keepdims=True)
            cent = x - mean
            var = jnp.mean(cent * cent, axis=-1, keepdims=True)
            xhat = cent * jax.lax.rsqrt(var + _EPS)
            o_ref[...] = xhat * gamma_ref[...] + beta_ref[...]

        buf = pl.Buffered(buffer_count=6)
        pipe = pltpu.emit_pipeline(
            inner,
            grid=(S // bs,),
            in_specs=[
                pl.BlockSpec((B, bs, H), lambda s: (0, s, 0), pipeline_mode=buf),
                pl.BlockSpec((bs, H), lambda s: (s, 0), pipeline_mode=buf),
            ],
            out_specs=[
                pl.BlockSpec((B, bs, H), lambda s: (0, s, 0),
                             pipeline_mode=pl.Buffered(buffer_count=2)),
            ],
        )
        pipe(x_hbm, pos_hbm, o_hbm)

    return pl.pallas_call(
        outer,
        in_specs=[
            pl.BlockSpec(memory_space=pl.ANY),
            pl.BlockSpec(memory_space=pl.ANY),
            pl.BlockSpec(memory_space=pltpu.VMEM),
            pl.BlockSpec(memory_space=pltpu.VMEM),
        ],
        out_specs=pl.BlockSpec(memory_space=pl.ANY),
        out_shape=jax.ShapeDtypeStruct((B, S, H), concat_embeddings.dtype),
        compiler_params=pltpu.CompilerParams(
            vmem_limit_bytes=128 * 1024 * 1024),
    )(concat_embeddings, pos_table, gamma, beta)
